# Initial kernel scaffold; baseline (speedup 1.0000x reference)
import jax
import jax.numpy as jnp
from jax.experimental import pallas as pl


def kernel(rays, radii):
    raise NotImplementedError("write your pallas kernel here")


# speedup vs baseline: 1.4367x; 1.4367x over previous
import functools

import jax
import jax.numpy as jnp
from jax.experimental import pallas as pl
from jax.experimental.pallas import tpu as pltpu

N = 65536
S = 128
ROWS = 256


def _roll(x, shift):
    return pltpu.roll(x, shift, 1)


def _body(rays_ref, radii_ref, out_ref, depth_ref, idx_ref, mask_ref):
    rays = rays_ref[...]
    o = rays[:, 0:3]
    d = rays[:, 3:6]
    a = jnp.sum(d * d, axis=1, keepdims=True)
    b = jnp.sum(o * d, axis=1, keepdims=True)
    c0 = jnp.sum(o * o, axis=1, keepdims=True)
    r = radii_ref[...]
    disc = b * b - a * (c0 - r * r)
    sq = jnp.sqrt(jnp.maximum(disc, 0.0))
    t = (sq - b) / a
    x = jnp.abs(t) * jnp.sqrt(a)

    lane = jax.lax.broadcasted_iota(jnp.int32, (ROWS, S), 1)
    for k in (64, 32, 16, 8, 4, 2, 1):
        upper = (lane & k) != 0
        partner = jnp.where(upper, _roll(x, k), _roll(x, S - k))
        x = jnp.where(upper, jnp.maximum(x, partner), jnp.minimum(x, partner))
    depth_ref[...] = x

    r1 = radii_ref[0, 1]
    v7 = jnp.concatenate([o / r1, d, jnp.full((ROWS, 1), r1, jnp.float32)], axis=1)
    out_ref[...] = jnp.broadcast_to(v7[:, None, :], (ROWS, S, 7))
    idx_ref[...] = jnp.ones((ROWS, S), jnp.int32)
    mask_ref[...] = jnp.zeros((ROWS, S), jnp.bool_)


@functools.partial(jax.jit, static_argnames=("interpret",))
def kernel(rays, radii, interpret=False):
    radii2d = radii.reshape(1, S)
    grid = (N // ROWS,)
    out, depth, idx, mask = pl.pallas_call(
        _body,
        grid=grid,
        in_specs=[
            pl.BlockSpec((ROWS, 6), lambda i: (i, 0)),
            pl.BlockSpec((1, S), lambda i: (0, 0)),
        ],
        out_specs=[
            pl.BlockSpec((ROWS, S, 7), lambda i: (i, 0, 0)),
            pl.BlockSpec((ROWS, S), lambda i: (i, 0)),
            pl.BlockSpec((ROWS, S), lambda i: (i, 0)),
            pl.BlockSpec((ROWS, S), lambda i: (i, 0)),
        ],
        out_shape=[
            jax.ShapeDtypeStruct((N, S, 7), jnp.float32),
            jax.ShapeDtypeStruct((N, S), jnp.float32),
            jax.ShapeDtypeStruct((N, S), jnp.int32),
            jax.ShapeDtypeStruct((N, S), jnp.bool_),
        ],
        interpret=interpret,
    )(rays, radii2d)
    return out, depth, idx, mask


# speedup vs baseline: 5.0389x; 3.5073x over previous
import functools

import jax
import jax.numpy as jnp
from jax.experimental import pallas as pl
from jax.experimental.pallas import tpu as pltpu

N = 65536
S = 128
ROWS = 256


def _roll(x, shift):
    return pltpu.roll(x, shift, 1)


def _body(rays_ref, radii_ref, out_ref, depth_ref, idx_ref, mask_ref):
    rays = rays_ref[...]
    o = rays[:, 0:3]
    d = rays[:, 3:6]
    a = jnp.sum(d * d, axis=1, keepdims=True)
    b = jnp.sum(o * d, axis=1, keepdims=True)
    c0 = jnp.sum(o * o, axis=1, keepdims=True)
    r = radii_ref[...]
    disc = b * b - a * (c0 - r * r)
    sq = jnp.sqrt(jnp.maximum(disc, 0.0))
    t = (sq - b) / a
    x = jnp.abs(t) * jnp.sqrt(a)

    lane = jax.lax.broadcasted_iota(jnp.int32, (ROWS, S), 1)
    for k in (64, 32, 16, 8, 4, 2, 1):
        upper = (lane & k) != 0
        partner = jnp.where(upper, _roll(x, k), _roll(x, S - k))
        x = jnp.where(upper, jnp.maximum(x, partner), jnp.minimum(x, partner))
    depth_ref[...] = x

    r1 = radii_ref[0, 1]
    mod7 = jax.lax.broadcasted_iota(jnp.int32, (ROWS, S * 7), 1) % 7
    vals = [o[:, 0:1] / r1, o[:, 1:2] / r1, o[:, 2:3] / r1,
            d[:, 0:1], d[:, 1:2], d[:, 2:3]]
    acc = jnp.full((ROWS, S * 7), r1, jnp.float32)
    for j in range(6):
        acc = jnp.where(mod7 == j, vals[j], acc)
    out_ref[...] = acc
    idx_ref[...] = jnp.ones((ROWS, S), jnp.int32)
    mask_ref[...] = jnp.zeros((ROWS, S), jnp.bool_)


@functools.partial(jax.jit, static_argnames=("interpret",))
def kernel(rays, radii, interpret=False):
    radii2d = radii.reshape(1, S)
    grid = (N // ROWS,)
    out, depth, idx, mask = pl.pallas_call(
        _body,
        grid=grid,
        in_specs=[
            pl.BlockSpec((ROWS, 6), lambda i: (i, 0)),
            pl.BlockSpec((1, S), lambda i: (0, 0)),
        ],
        out_specs=[
            pl.BlockSpec((ROWS, S * 7), lambda i: (i, 0)),
            pl.BlockSpec((ROWS, S), lambda i: (i, 0)),
            pl.BlockSpec((ROWS, S), lambda i: (i, 0)),
            pl.BlockSpec((ROWS, S), lambda i: (i, 0)),
        ],
        out_shape=[
            jax.ShapeDtypeStruct((N, S * 7), jnp.float32),
            jax.ShapeDtypeStruct((N, S), jnp.float32),
            jax.ShapeDtypeStruct((N, S), jnp.int32),
            jax.ShapeDtypeStruct((N, S), jnp.bool_),
        ],
        interpret=interpret,
    )(rays, radii2d)
    return out.reshape(N, S, 7), depth, idx, mask


# speedup vs baseline: 5.0543x; 1.0031x over previous
import functools

import jax
import jax.numpy as jnp
from jax.experimental import pallas as pl
from jax.experimental.pallas import tpu as pltpu

N = 65536
S = 128
ROWS = 256


def _roll(x, shift):
    return pltpu.roll(x, shift, 1)


def _body(rays_ref, radii_ref, out_ref, depth_ref, idx_ref, mask_ref):
    rays = rays_ref[...]
    o = rays[:, 0:3]
    d = rays[:, 3:6]
    a = jnp.sum(d * d, axis=1, keepdims=True)
    b = jnp.sum(o * d, axis=1, keepdims=True)
    c0 = jnp.sum(o * o, axis=1, keepdims=True)
    r = radii_ref[...]
    disc = b * b - a * (c0 - r * r)
    sq = jnp.sqrt(jnp.maximum(disc, 0.0))
    t = (sq - b) / a
    x = jnp.abs(t) * jnp.sqrt(a)

    lane = jax.lax.broadcasted_iota(jnp.int32, (ROWS, S), 1)
    for k in (64, 32, 16, 8, 4, 2, 1):
        upper = (lane & k) != 0
        partner = jnp.where(upper, _roll(x, k), _roll(x, S - k))
        x = jnp.where(upper, jnp.maximum(x, partner), jnp.minimum(x, partner))
    depth_ref[...] = x

    r1 = radii_ref[0, 1]
    mod7 = jax.lax.broadcasted_iota(jnp.int32, (ROWS, S * 7), 1) % 7
    vals = [o[:, 0:1] / r1, o[:, 1:2] / r1, o[:, 2:3] / r1,
            d[:, 0:1], d[:, 1:2], d[:, 2:3]]
    acc = jnp.full((ROWS, S * 7), r1, jnp.float32)
    for j in range(6):
        acc = jnp.where(mod7 == j, vals[j], acc)
    out_ref[...] = acc
    idx_ref[...] = jnp.ones((ROWS, S), jnp.int32)
    mask_ref[...] = jnp.zeros((ROWS, S), jnp.bool_)


@functools.partial(jax.jit, static_argnames=("interpret",))
def kernel(rays, radii, interpret=False):
    radii2d = radii.reshape(1, S)
    grid = (N // ROWS,)
    out, depth, idx, mask = pl.pallas_call(
        _body,
        grid=grid,
        in_specs=[
            pl.BlockSpec((ROWS, 6), lambda i: (i, 0)),
            pl.BlockSpec((1, S), lambda i: (0, 0)),
        ],
        out_specs=[
            pl.BlockSpec((ROWS, S * 7), lambda i: (i, 0)),
            pl.BlockSpec((ROWS, S), lambda i: (i, 0)),
            pl.BlockSpec((ROWS, S), lambda i: (i, 0)),
            pl.BlockSpec((ROWS, S), lambda i: (i, 0)),
        ],
        out_shape=[
            jax.ShapeDtypeStruct((N, S * 7), jnp.float32),
            jax.ShapeDtypeStruct((N, S), jnp.float32),
            jax.ShapeDtypeStruct((N, S), jnp.int32),
            jax.ShapeDtypeStruct((N, S), jnp.bool_),
        ],
        compiler_params=pltpu.CompilerParams(
            dimension_semantics=("parallel",),
        ),
        interpret=interpret,
    )(rays, radii2d)
    return out.reshape(N, S, 7), depth, idx, mask


# speedup vs baseline: 6.3495x; 1.2562x over previous
import functools

import jax
import jax.numpy as jnp
from jax.experimental import pallas as pl
from jax.experimental.pallas import tpu as pltpu

N = 65536
S = 128
ROWS = 1024


def _roll(x, shift):
    return pltpu.roll(x, shift, 1)


def _body(rays_ref, radii_ref, out_ref, depth_ref, idx_ref, mask_ref):
    rays = rays_ref[...]
    o = rays[:, 0:3]
    d = rays[:, 3:6]
    a = jnp.sum(d * d, axis=1, keepdims=True)
    b = jnp.sum(o * d, axis=1, keepdims=True)
    c0 = jnp.sum(o * o, axis=1, keepdims=True)
    r = radii_ref[...]
    disc = b * b - a * (c0 - r * r)
    sq = jnp.sqrt(jnp.maximum(disc, 0.0))
    t = (sq - b) / a
    x = jnp.abs(t) * jnp.sqrt(a)

    lane = jax.lax.broadcasted_iota(jnp.int32, (ROWS, S), 1)
    for k in (64, 32, 16, 8, 4, 2, 1):
        upper = (lane & k) != 0
        partner = jnp.where(upper, _roll(x, k), _roll(x, S - k))
        x = jnp.where(upper, jnp.maximum(x, partner), jnp.minimum(x, partner))
    depth_ref[...] = x

    r1 = radii_ref[0, 1]
    vals = jnp.concatenate(
        [o / r1, d, jnp.full((ROWS, 1), r1, jnp.float32),
         jnp.zeros((ROWS, 1), jnp.float32)], axis=1)
    hi = vals.astype(jnp.bfloat16)
    lo = (vals - hi.astype(jnp.float32)).astype(jnp.bfloat16)
    mod7 = jax.lax.broadcasted_iota(jnp.int32, (8, S * 7), 1) % 7
    row8 = jax.lax.broadcasted_iota(jnp.int32, (8, S * 7), 0)
    p = (mod7 == row8).astype(jnp.bfloat16)
    out_ref[...] = (
        jnp.dot(hi, p, preferred_element_type=jnp.float32)
        + jnp.dot(lo, p, preferred_element_type=jnp.float32))
    idx_ref[...] = jnp.ones((ROWS, S), jnp.int32)
    mask_ref[...] = jnp.zeros((ROWS, S), jnp.bool_)


@functools.partial(jax.jit, static_argnames=("interpret",))
def kernel(rays, radii, interpret=False):
    radii2d = radii.reshape(1, S)
    grid = (N // ROWS,)
    out, depth, idx, mask = pl.pallas_call(
        _body,
        grid=grid,
        in_specs=[
            pl.BlockSpec((ROWS, 6), lambda i: (i, 0)),
            pl.BlockSpec((1, S), lambda i: (0, 0)),
        ],
        out_specs=[
            pl.BlockSpec((ROWS, S * 7), lambda i: (i, 0)),
            pl.BlockSpec((ROWS, S), lambda i: (i, 0)),
            pl.BlockSpec((ROWS, S), lambda i: (i, 0)),
            pl.BlockSpec((ROWS, S), lambda i: (i, 0)),
        ],
        out_shape=[
            jax.ShapeDtypeStruct((N, S * 7), jnp.float32),
            jax.ShapeDtypeStruct((N, S), jnp.float32),
            jax.ShapeDtypeStruct((N, S), jnp.int32),
            jax.ShapeDtypeStruct((N, S), jnp.bool_),
        ],
        compiler_params=pltpu.CompilerParams(
            dimension_semantics=("parallel",),
        ),
        interpret=interpret,
    )(rays, radii2d)
    return out.reshape(N, S, 7), depth, idx, mask


# speedup vs baseline: 16.5430x; 2.6054x over previous
import functools

import jax
import jax.numpy as jnp
from jax.experimental import pallas as pl
from jax.experimental.pallas import tpu as pltpu

N = 65536
S = 128
ROWS = 1024


def _roll(x, shift):
    return pltpu.roll(x, shift, 1)


def _body(rays_ref, radii_ref, vals_ref, depth_ref, idx_ref, mask_ref):
    rays = rays_ref[...]
    o = rays[:, 0:3]
    d = rays[:, 3:6]
    a = jnp.sum(d * d, axis=1, keepdims=True)
    b = jnp.sum(o * d, axis=1, keepdims=True)
    c0 = jnp.sum(o * o, axis=1, keepdims=True)
    r = radii_ref[...]
    disc = b * b - a * (c0 - r * r)
    sq = jnp.sqrt(jnp.maximum(disc, 0.0))
    t = (sq - b) / a
    x = jnp.abs(t) * jnp.sqrt(a)

    lane = jax.lax.broadcasted_iota(jnp.int32, (1, S), 1)
    for k in (64, 32, 16, 8, 4, 2, 1):
        upper = (lane & k) != 0
        u = _roll(x, S - k)
        v = _roll(x, k)
        x = jnp.where(upper, jnp.maximum(x, v), jnp.minimum(x, u))
    depth_ref[...] = x

    r1 = radii_ref[0, 1]
    vals_ref[...] = jnp.concatenate(
        [o / r1, d, jnp.full((ROWS, 2), r1, jnp.float32)], axis=1)
    idx_ref[...] = jnp.ones((ROWS, S), jnp.int32)
    mask_ref[...] = jnp.zeros((ROWS, S), jnp.bool_)


@functools.partial(jax.jit, static_argnames=("interpret",))
def kernel(rays, radii, interpret=False):
    radii2d = radii.reshape(1, S)
    grid = (N // ROWS,)
    vals, depth, idx, mask = pl.pallas_call(
        _body,
        grid=grid,
        in_specs=[
            pl.BlockSpec((ROWS, 6), lambda i: (i, 0)),
            pl.BlockSpec((1, S), lambda i: (0, 0)),
        ],
        out_specs=[
            pl.BlockSpec((ROWS, 8), lambda i: (i, 0)),
            pl.BlockSpec((ROWS, S), lambda i: (i, 0)),
            pl.BlockSpec((ROWS, S), lambda i: (i, 0)),
            pl.BlockSpec((ROWS, S), lambda i: (i, 0)),
        ],
        out_shape=[
            jax.ShapeDtypeStruct((N, 8), jnp.float32),
            jax.ShapeDtypeStruct((N, S), jnp.float32),
            jax.ShapeDtypeStruct((N, S), jnp.int32),
            jax.ShapeDtypeStruct((N, S), jnp.bool_),
        ],
        compiler_params=pltpu.CompilerParams(
            dimension_semantics=("parallel",),
        ),
        interpret=interpret,
    )(rays, radii2d)
    out = jnp.broadcast_to(vals[:, None, 0:7], (N, S, 7))
    return out, depth, idx, mask


# speedup vs baseline: 16.8462x; 1.0183x over previous
import functools

import jax
import jax.numpy as jnp
from jax.experimental import pallas as pl
from jax.experimental.pallas import tpu as pltpu

N = 65536
S = 128
ROWS = 1024
CH = 128


def _roll(x, shift):
    return pltpu.roll(x, shift, 1)


def _body(rays_ref, radii_ref, vals_ref, depth_ref, idx_ref, mask_ref):
    r = radii_ref[...]
    r2 = r * r
    r1 = radii_ref[0, 1]
    lane = jax.lax.broadcasted_iota(jnp.int32, (1, S), 1)

    rays = rays_ref[...]
    o = rays[:, 0:3]
    d = rays[:, 3:6]
    a = jnp.sum(d * d, axis=1, keepdims=True)
    b = jnp.sum(o * d, axis=1, keepdims=True)
    c0 = jnp.sum(o * o, axis=1, keepdims=True)
    e = b * b - a * c0
    isa = jax.lax.rsqrt(a)
    disc = a * r2 + e
    sq = jnp.sqrt(jnp.maximum(disc, 0.0))
    x = jnp.abs(sq - b) * isa

    for k in (64, 32, 16, 8, 4, 2, 1):
        upper = (lane & k) != 0
        u = _roll(x, S - k)
        v = _roll(x, k)
        x = jnp.where(upper, jnp.maximum(x, v), jnp.minimum(x, u))
    depth_ref[...] = x
    vals_ref[...] = jnp.concatenate(
        [o / r1, d, jnp.full((ROWS, 1), r1, jnp.float32)], axis=1)
    idx_ref[...] = jnp.ones((ROWS, S), jnp.int32)
    mask_ref[...] = jnp.zeros((ROWS, S), jnp.bool_)


@functools.partial(jax.jit, static_argnames=("interpret",))
def kernel(rays, radii, interpret=False):
    radii2d = radii.reshape(1, S)
    grid = (N // ROWS,)
    vals, depth, idx, mask = pl.pallas_call(
        _body,
        grid=grid,
        in_specs=[
            pl.BlockSpec((ROWS, 6), lambda i: (i, 0)),
            pl.BlockSpec((1, S), lambda i: (0, 0)),
        ],
        out_specs=[
            pl.BlockSpec((ROWS, 7), lambda i: (i, 0)),
            pl.BlockSpec((ROWS, S), lambda i: (i, 0)),
            pl.BlockSpec((ROWS, S), lambda i: (i, 0)),
            pl.BlockSpec((ROWS, S), lambda i: (i, 0)),
        ],
        out_shape=[
            jax.ShapeDtypeStruct((N, 7), jnp.float32),
            jax.ShapeDtypeStruct((N, S), jnp.float32),
            jax.ShapeDtypeStruct((N, S), jnp.int32),
            jax.ShapeDtypeStruct((N, S), jnp.bool_),
        ],
        compiler_params=pltpu.CompilerParams(
            dimension_semantics=("parallel",),
        ),
        interpret=interpret,
    )(rays, radii2d)
    out = jnp.broadcast_to(vals[:, None, :], (N, S, 7))
    return out, depth, idx, mask


# speedup vs baseline: 17.9464x; 1.0653x over previous
import functools

import jax
import jax.numpy as jnp
from jax.experimental import pallas as pl
from jax.experimental.pallas import tpu as pltpu

N = 65536
S = 128
ROWS = 1024
CH = 128


def _roll(x, shift):
    return pltpu.roll(x, shift, 1)


def _body(rays_ref, radii_ref, vals_ref, depth_ref, idx_ref, mask_ref):
    r = radii_ref[...]
    r2 = r * r
    r1 = radii_ref[0, 1]
    lane = jax.lax.broadcasted_iota(jnp.int32, (1, S), 1)

    rays = rays_ref[...]
    o = rays[:, 0:3]
    d = rays[:, 3:6]
    a = jnp.sum(d * d, axis=1, keepdims=True)
    b = jnp.sum(o * d, axis=1, keepdims=True)
    c0 = jnp.sum(o * o, axis=1, keepdims=True)
    e = b * b - a * c0
    isa = jax.lax.rsqrt(a)
    disc = a * r2 + e
    sq = jnp.sqrt(jnp.maximum(disc, 0.0))
    x = jnp.abs(sq - b) * isa

    for k in (64, 32, 16, 8, 4, 2, 1):
        upper = (lane & k) != 0
        u = _roll(x, S - k)
        v = _roll(x, k)
        x = jnp.where(upper, jnp.maximum(x, v), jnp.minimum(x, u))
    depth_ref[...] = x
    vals = jnp.concatenate(
        [o / r1, d, jnp.full((ROWS, 2), r1, jnp.float32)], axis=1)
    vals_ref[...] = vals.T
    idx_ref[...] = jnp.ones((ROWS, S), jnp.int32)
    mask_ref[...] = jnp.zeros((ROWS, S), jnp.bool_)


@functools.partial(jax.jit, static_argnames=("interpret",))
def kernel(rays, radii, interpret=False):
    radii2d = radii.reshape(1, S)
    grid = (N // ROWS,)
    vals, depth, idx, mask = pl.pallas_call(
        _body,
        grid=grid,
        in_specs=[
            pl.BlockSpec((ROWS, 6), lambda i: (i, 0)),
            pl.BlockSpec((1, S), lambda i: (0, 0)),
        ],
        out_specs=[
            pl.BlockSpec((8, ROWS), lambda i: (0, i)),
            pl.BlockSpec((ROWS, S), lambda i: (i, 0)),
            pl.BlockSpec((ROWS, S), lambda i: (i, 0)),
            pl.BlockSpec((ROWS, S), lambda i: (i, 0)),
        ],
        out_shape=[
            jax.ShapeDtypeStruct((8, N), jnp.float32),
            jax.ShapeDtypeStruct((N, S), jnp.float32),
            jax.ShapeDtypeStruct((N, S), jnp.int32),
            jax.ShapeDtypeStruct((N, S), jnp.bool_),
        ],
        compiler_params=pltpu.CompilerParams(
            dimension_semantics=("parallel",),
        ),
        interpret=interpret,
    )(rays, radii2d)
    out = jnp.broadcast_to(vals.T[:, None, 0:7], (N, S, 7))
    return out, depth, idx, mask


# speedup vs baseline: 18.0581x; 1.0062x over previous
import functools

import jax
import jax.numpy as jnp
from jax.experimental import pallas as pl
from jax.experimental.pallas import tpu as pltpu

N = 65536
S = 128
ROWS = 2048
CH = 128


def _roll(x, shift):
    return pltpu.roll(x, shift, 1)


def _body(rays_ref, radii_ref, vals_ref, depth_ref, idx_ref, mask_ref):
    r = radii_ref[...]
    r2 = r * r
    r1 = radii_ref[0, 1]
    lane = jax.lax.broadcasted_iota(jnp.int32, (1, S), 1)

    rays = rays_ref[...]
    o = rays[:, 0:3]
    d = rays[:, 3:6]
    a = jnp.sum(d * d, axis=1, keepdims=True)
    b = jnp.sum(o * d, axis=1, keepdims=True)
    c0 = jnp.sum(o * o, axis=1, keepdims=True)
    e = b * b - a * c0
    isa = jax.lax.rsqrt(a)
    disc = a * r2 + e
    sq = jnp.sqrt(jnp.maximum(disc, 0.0))
    x = jnp.abs(sq - b) * isa

    for k in (64, 32, 16, 8, 4, 2, 1):
        upper = (lane & k) != 0
        u = _roll(x, S - k)
        v = _roll(x, k)
        x = jnp.where(upper, jnp.maximum(x, v), jnp.minimum(x, u))
    depth_ref[...] = x
    vals = jnp.concatenate(
        [o / r1, d, jnp.full((ROWS, 2), r1, jnp.float32)], axis=1)
    vals_ref[...] = vals.T
    idx_ref[...] = jnp.ones((ROWS, S), jnp.int32)
    mask_ref[...] = jnp.zeros((ROWS, S), jnp.bool_)


@functools.partial(jax.jit, static_argnames=("interpret",))
def kernel(rays, radii, interpret=False):
    radii2d = radii.reshape(1, S)
    grid = (N // ROWS,)
    vals, depth, idx, mask = pl.pallas_call(
        _body,
        grid=grid,
        in_specs=[
            pl.BlockSpec((ROWS, 6), lambda i: (i, 0)),
            pl.BlockSpec((1, S), lambda i: (0, 0)),
        ],
        out_specs=[
            pl.BlockSpec((8, ROWS), lambda i: (0, i)),
            pl.BlockSpec((ROWS, S), lambda i: (i, 0)),
            pl.BlockSpec((ROWS, S), lambda i: (i, 0)),
            pl.BlockSpec((ROWS, S), lambda i: (i, 0)),
        ],
        out_shape=[
            jax.ShapeDtypeStruct((8, N), jnp.float32),
            jax.ShapeDtypeStruct((N, S), jnp.float32),
            jax.ShapeDtypeStruct((N, S), jnp.int32),
            jax.ShapeDtypeStruct((N, S), jnp.bool_),
        ],
        compiler_params=pltpu.CompilerParams(
            dimension_semantics=("parallel",),
        ),
        interpret=interpret,
    )(rays, radii2d)
    out = jnp.broadcast_to(vals.T[:, None, 0:7], (N, S, 7))
    return out, depth, idx, mask


# speedup vs baseline: 19.8671x; 1.1002x over previous
import functools

import jax
import jax.numpy as jnp
from jax.experimental import pallas as pl
from jax.experimental.pallas import tpu as pltpu

N = 65536
S = 128
ROWS = 2048


def _roll(x, shift):
    return pltpu.roll(x, shift, 1)


def _body(rays_ref, radii_ref, vals_ref, depth_ref, idx_ref, mask_ref):
    r = radii_ref[...]
    r2 = r * r
    r1 = radii_ref[0, 1]
    lane = jax.lax.broadcasted_iota(jnp.int32, (1, S), 1)

    rt = rays_ref[...]
    ox, oy, oz = rt[0:1, :], rt[1:2, :], rt[2:3, :]
    dx, dy, dz = rt[3:4, :], rt[4:5, :], rt[5:6, :]
    a_r = dx * dx + dy * dy + dz * dz
    b_r = ox * dx + oy * dy + oz * dz
    c0_r = ox * ox + oy * oy + oz * oz
    e_r = b_r * b_r - a_r * c0_r
    isa_r = jax.lax.rsqrt(a_r)
    zero = jnp.zeros((4, ROWS), jnp.float32)
    cols = jnp.concatenate([a_r, b_r, e_r, isa_r, zero], axis=0).T
    a = cols[:, 0:1]
    b = cols[:, 1:2]
    e = cols[:, 2:3]
    isa = cols[:, 3:4]

    disc = a * r2 + e
    sq = jnp.sqrt(jnp.maximum(disc, 0.0))
    x = jnp.abs(sq - b) * isa

    for k in (64, 32, 16, 8, 4, 2, 1):
        upper = (lane & k) != 0
        u = _roll(x, S - k)
        v = _roll(x, k)
        x = jnp.where(upper, jnp.maximum(x, v), jnp.minimum(x, u))
    depth_ref[...] = x

    inv_r1 = 1.0 / r1
    r1row = jnp.full((2, ROWS), r1, jnp.float32)
    vals_ref[...] = jnp.concatenate(
        [ox * inv_r1, oy * inv_r1, oz * inv_r1, dx, dy, dz, r1row],
        axis=0)
    idx_ref[...] = jnp.ones((ROWS, S), jnp.int32)
    mask_ref[...] = jnp.zeros((ROWS, S), jnp.bool_)


@functools.partial(jax.jit, static_argnames=("interpret",))
def kernel(rays, radii, interpret=False):
    radii2d = radii.reshape(1, S)
    rays_t = rays.T
    grid = (N // ROWS,)
    vals, depth, idx, mask = pl.pallas_call(
        _body,
        grid=grid,
        in_specs=[
            pl.BlockSpec((6, ROWS), lambda i: (0, i)),
            pl.BlockSpec((1, S), lambda i: (0, 0)),
        ],
        out_specs=[
            pl.BlockSpec((8, ROWS), lambda i: (0, i)),
            pl.BlockSpec((ROWS, S), lambda i: (i, 0)),
            pl.BlockSpec((ROWS, S), lambda i: (i, 0)),
            pl.BlockSpec((ROWS, S), lambda i: (i, 0)),
        ],
        out_shape=[
            jax.ShapeDtypeStruct((8, N), jnp.float32),
            jax.ShapeDtypeStruct((N, S), jnp.float32),
            jax.ShapeDtypeStruct((N, S), jnp.int32),
            jax.ShapeDtypeStruct((N, S), jnp.bool_),
        ],
        compiler_params=pltpu.CompilerParams(
            dimension_semantics=("parallel",),
        ),
        interpret=interpret,
    )(rays_t, radii2d)
    out = jnp.broadcast_to(vals.T[:, None, 0:7], (N, S, 7))
    return out, depth, idx, mask


# speedup vs baseline: 20.0122x; 1.0073x over previous
import functools

import jax
import jax.numpy as jnp
from jax.experimental import pallas as pl
from jax.experimental.pallas import tpu as pltpu

N = 65536
S = 128
ROWS = 4096


def _roll(x, shift):
    return pltpu.roll(x, shift, 1)


def _body(rays_ref, radii_ref, vals_ref, depth_ref, idx_ref, mask_ref):
    r = radii_ref[...]
    r2 = r * r
    r1 = radii_ref[0, 1]
    lane = jax.lax.broadcasted_iota(jnp.int32, (1, S), 1)

    rt = rays_ref[...]
    ox, oy, oz = rt[0:1, :], rt[1:2, :], rt[2:3, :]
    dx, dy, dz = rt[3:4, :], rt[4:5, :], rt[5:6, :]
    a_r = dx * dx + dy * dy + dz * dz
    b_r = ox * dx + oy * dy + oz * dz
    c0_r = ox * ox + oy * oy + oz * oz
    e_r = b_r * b_r - a_r * c0_r
    isa_r = jax.lax.rsqrt(a_r)
    zero = jnp.zeros((4, ROWS), jnp.float32)
    cols = jnp.concatenate([a_r, b_r, e_r, isa_r, zero], axis=0).T
    a = cols[:, 0:1]
    b = cols[:, 1:2]
    e = cols[:, 2:3]
    isa = cols[:, 3:4]

    disc = a * r2 + e
    sq = jnp.sqrt(jnp.maximum(disc, 0.0))
    x = jnp.abs(sq - b) * isa

    for k in (64, 32, 16, 8, 4, 2, 1):
        upper = (lane & k) != 0
        u = _roll(x, S - k)
        v = _roll(x, k)
        x = jnp.where(upper, jnp.maximum(x, v), jnp.minimum(x, u))
    depth_ref[...] = x

    inv_r1 = 1.0 / r1
    r1row = jnp.full((2, ROWS), r1, jnp.float32)
    vals_ref[...] = jnp.concatenate(
        [ox * inv_r1, oy * inv_r1, oz * inv_r1, dx, dy, dz, r1row],
        axis=0)
    idx_ref[...] = jnp.ones((ROWS, S), jnp.int32)
    mask_ref[...] = jnp.zeros((ROWS, S), jnp.bool_)


@functools.partial(jax.jit, static_argnames=("interpret",))
def kernel(rays, radii, interpret=False):
    radii2d = radii.reshape(1, S)
    rays_t = rays.T
    grid = (N // ROWS,)
    vals, depth, idx, mask = pl.pallas_call(
        _body,
        grid=grid,
        in_specs=[
            pl.BlockSpec((6, ROWS), lambda i: (0, i)),
            pl.BlockSpec((1, S), lambda i: (0, 0)),
        ],
        out_specs=[
            pl.BlockSpec((8, ROWS), lambda i: (0, i)),
            pl.BlockSpec((ROWS, S), lambda i: (i, 0)),
            pl.BlockSpec((ROWS, S), lambda i: (i, 0)),
            pl.BlockSpec((ROWS, S), lambda i: (i, 0)),
        ],
        out_shape=[
            jax.ShapeDtypeStruct((8, N), jnp.float32),
            jax.ShapeDtypeStruct((N, S), jnp.float32),
            jax.ShapeDtypeStruct((N, S), jnp.int32),
            jax.ShapeDtypeStruct((N, S), jnp.bool_),
        ],
        compiler_params=pltpu.CompilerParams(
            dimension_semantics=("parallel",),
        ),
        interpret=interpret,
    )(rays_t, radii2d)
    out = jnp.broadcast_to(vals.T[:, None, 0:7], (N, S, 7))
    return out, depth, idx, mask


# speedup vs baseline: 20.0126x; 1.0000x over previous
import jax
import jax.numpy as jnp
from jax.experimental import pallas as pl
from jax.experimental.pallas import tpu as pltpu

N = 65536
S = 128
ROWS = 4096


def _roll(x, shift):
    return pltpu.roll(x, shift, 1)


def _body(rays_ref, radii_ref, vals_ref, depth_ref, idx_ref, mask_ref):
    r = radii_ref[...]
    r2 = r * r
    r1 = radii_ref[0, 1]
    lane = jax.lax.broadcasted_iota(jnp.int32, (1, S), 1)

    rt = rays_ref[...]
    ox, oy, oz = rt[0:1, :], rt[1:2, :], rt[2:3, :]
    dx, dy, dz = rt[3:4, :], rt[4:5, :], rt[5:6, :]
    a_r = dx * dx + dy * dy + dz * dz
    b_r = ox * dx + oy * dy + oz * dz
    c0_r = ox * ox + oy * oy + oz * oz
    e_r = b_r * b_r - a_r * c0_r
    isa_r = jax.lax.rsqrt(a_r)
    zero = jnp.zeros((4, ROWS), jnp.float32)
    cols = jnp.concatenate([a_r, b_r, e_r, isa_r, zero], axis=0).T
    a = cols[:, 0:1]
    b = cols[:, 1:2]
    e = cols[:, 2:3]
    isa = cols[:, 3:4]

    disc = a * r2 + e
    sq = jnp.sqrt(jnp.maximum(disc, 0.0))
    x = jnp.abs(sq - b) * isa

    for k in (64, 32, 16, 8, 4, 2, 1):
        upper = (lane & k) != 0
        u = _roll(x, S - k)
        v = _roll(x, k)
        x = jnp.where(upper, jnp.maximum(x, v), jnp.minimum(x, u))
    depth_ref[...] = x

    inv_r1 = 1.0 / r1
    r1row = jnp.full((2, ROWS), r1, jnp.float32)
    vals_ref[...] = jnp.concatenate(
        [ox * inv_r1, oy * inv_r1, oz * inv_r1, dx, dy, dz, r1row],
        axis=0)
    idx_ref[...] = jnp.ones((ROWS, S), jnp.int32)
    mask_ref[...] = jnp.zeros((ROWS, S), jnp.bool_)


@jax.jit
def kernel(rays, radii):
    radii2d = radii.reshape(1, S)
    rays_t = rays.T
    grid = (N // ROWS,)
    vals, depth, idx, mask = pl.pallas_call(
        _body,
        grid=grid,
        in_specs=[
            pl.BlockSpec((6, ROWS), lambda i: (0, i)),
            pl.BlockSpec((1, S), lambda i: (0, 0)),
        ],
        out_specs=[
            pl.BlockSpec((8, ROWS), lambda i: (0, i)),
            pl.BlockSpec((ROWS, S), lambda i: (i, 0)),
            pl.BlockSpec((ROWS, S), lambda i: (i, 0)),
            pl.BlockSpec((ROWS, S), lambda i: (i, 0)),
        ],
        out_shape=[
            jax.ShapeDtypeStruct((8, N), jnp.float32),
            jax.ShapeDtypeStruct((N, S), jnp.float32),
            jax.ShapeDtypeStruct((N, S), jnp.int32),
            jax.ShapeDtypeStruct((N, S), jnp.bool_),
        ],
        compiler_params=pltpu.CompilerParams(
            dimension_semantics=("parallel",),
        ),
    )(rays_t, radii2d)
    out = jnp.broadcast_to(vals.T[:, None, 0:7], (N, S, 7))
    return out, depth, idx, mask
